# baseline (speedup 1.0000x reference)
import functools
import math

import jax
import jax.numpy as jnp
from jax import lax
from jax.experimental import pallas as pl
from jax.experimental.pallas import tpu as pltpu
from jax.experimental.pallas import tpu_sc as plsc

N = 10000
E = 160000
HID = 128
K1 = int(math.ceil(0.8 * N))
K2 = int(math.ceil(0.8 * K1))

NP1, NP2, NP3 = 10240, 8192, 6656
NW = 32
CH = 128
NCH = 40
EPAD = NW * NCH * CH


def _row_chunks(rpt, step=128):
    out, off = [], 0
    while off < rpt:
        sz = min(step, rpt - off)
        out.append((off, sz))
        off += sz
    return out


def _make_agg(n_pad):
    rpt = n_pad // 16
    zchunks = _row_chunks(rpt, 16)
    chunks = _row_chunks(rpt)
    mesh = plsc.VectorSubcoreMesh(core_axis_name="c", subcore_axis_name="s")

    @functools.partial(
        pl.kernel,
        out_type=jax.ShapeDtypeStruct((2, n_pad, HID), jnp.float32),
        mesh=mesh,
        scratch_types=[
            pltpu.VMEM((NCH, CH), jnp.int32),
            pltpu.VMEM((NCH, CH), jnp.int32),
            pltpu.VMEM((2, CH, HID), jnp.float32),
            pltpu.VMEM((16, HID), jnp.float32),
            pltpu.VMEM_SHARED((n_pad, HID), jnp.float32),
            pltpu.SemaphoreType.DMA,
            pltpu.SemaphoreType.DMA,
        ],
    )
    def agg(h_hbm, srcp_hbm, dstp_hbm, z_hbm, out_hbm,
            src_v, dst_v, rows_v, zbuf, acc, sem0, sem1):
        c = lax.axis_index("c")
        s = lax.axis_index("s")
        wid = c * 16 + s
        row0 = s * rpt
        pltpu.sync_copy(z_hbm, zbuf)
        for off, sz in zchunks:
            pltpu.sync_copy(zbuf.at[pl.ds(0, sz)], acc.at[pl.ds(row0 + off, sz)])
        pltpu.sync_copy(srcp_hbm.at[wid], src_v)
        pltpu.sync_copy(dstp_hbm.at[wid], dst_v)
        plsc.subcore_barrier()
        sems = (sem0, sem1)
        cps = [None, None]
        cps[0] = pltpu.async_copy(h_hbm.at[src_v.at[0]], rows_v.at[0], sem0)
        for j in range(NCH):
            b = j % 2
            cps[b].wait()
            if j + 1 < NCH:
                nb = (j + 1) % 2
                cps[nb] = pltpu.async_copy(h_hbm.at[src_v.at[j + 1]],
                                           rows_v.at[nb], sems[nb])
            pltpu.sync_copy(rows_v.at[b], acc.at[dst_v.at[j]], add=True)
        plsc.subcore_barrier()
        for off, sz in chunks:
            pltpu.sync_copy(acc.at[pl.ds(row0 + off, sz)],
                            out_hbm.at[c, pl.ds(row0 + off, sz)])

    return agg


def _make_deg(n_pad):
    rpt = n_pad // 16
    zchunks = _row_chunks(rpt, 16)
    chunks = _row_chunks(rpt)
    mesh = plsc.VectorSubcoreMesh(core_axis_name="c", subcore_axis_name="s")

    @functools.partial(
        pl.kernel,
        out_type=jax.ShapeDtypeStruct((2, n_pad, HID), jnp.float32),
        mesh=mesh,
        scratch_types=[
            pltpu.VMEM((NCH, CH), jnp.int32),
            pltpu.VMEM((CH, HID), jnp.float32),
            pltpu.VMEM((16, HID), jnp.float32),
            pltpu.VMEM_SHARED((n_pad, HID), jnp.float32),
        ],
    )
    def deg(ones_hbm, z_hbm, dstp_hbm, out_hbm, dst_v, ones_v, zbuf, acc):
        c = lax.axis_index("c")
        s = lax.axis_index("s")
        wid = c * 16 + s
        row0 = s * rpt
        pltpu.sync_copy(z_hbm, zbuf)
        for off, sz in zchunks:
            pltpu.sync_copy(zbuf.at[pl.ds(0, sz)], acc.at[pl.ds(row0 + off, sz)])
        pltpu.sync_copy(ones_hbm, ones_v)
        pltpu.sync_copy(dstp_hbm.at[wid], dst_v)
        plsc.subcore_barrier()
        for j in range(NCH):
            pltpu.sync_copy(ones_v, acc.at[dst_v.at[j]], add=True)
        plsc.subcore_barrier()
        for off, sz in chunks:
            pltpu.sync_copy(acc.at[pl.ds(row0 + off, sz)],
                            out_hbm.at[c, pl.ds(row0 + off, sz)])

    return deg


_BR = 512


def _mm_body(x_ref, w_ref, b_ref, o_ref):
    o_ref[...] = jnp.dot(x_ref[...], w_ref[...],
                         preferred_element_type=jnp.float32) + b_ref[...]


def _mm_nob_body(x_ref, w_ref, o_ref):
    o_ref[...] = jnp.dot(x_ref[...], w_ref[...],
                         preferred_element_type=jnp.float32)


def _mm_prelu_body(x_ref, w_ref, b_ref, a_ref, o_ref):
    y = jnp.dot(x_ref[...], w_ref[...],
                preferred_element_type=jnp.float32) + b_ref[...]
    a = a_ref[0]
    o_ref[...] = jnp.where(y > 0, y, a * y)


def _mm(x, w, b):
    n, k = x.shape
    return pl.pallas_call(
        _mm_body,
        grid=(n // _BR,),
        in_specs=[
            pl.BlockSpec((_BR, k), lambda i: (i, 0)),
            pl.BlockSpec((k, HID), lambda i: (0, 0)),
            pl.BlockSpec((1, HID), lambda i: (0, 0)),
        ],
        out_specs=pl.BlockSpec((_BR, HID), lambda i: (i, 0)),
        out_shape=jax.ShapeDtypeStruct((n, HID), jnp.float32),
    )(x, w, b.reshape(1, HID))


def _mm_nob(x, w):
    n, k = x.shape
    return pl.pallas_call(
        _mm_nob_body,
        grid=(n // _BR,),
        in_specs=[
            pl.BlockSpec((_BR, k), lambda i: (i, 0)),
            pl.BlockSpec((k, HID), lambda i: (0, 0)),
        ],
        out_specs=pl.BlockSpec((_BR, HID), lambda i: (i, 0)),
        out_shape=jax.ShapeDtypeStruct((n, HID), jnp.float32),
    )(x, w)


def _mm_prelu(x, w, b, a):
    n, k = x.shape
    return pl.pallas_call(
        _mm_prelu_body,
        grid=(n // _BR,),
        in_specs=[
            pl.BlockSpec((_BR, k), lambda i: (i, 0)),
            pl.BlockSpec((k, HID), lambda i: (0, 0)),
            pl.BlockSpec((1, HID), lambda i: (0, 0)),
            pl.BlockSpec(memory_space=pltpu.SMEM),
        ],
        out_specs=pl.BlockSpec((_BR, HID), lambda i: (i, 0)),
        out_shape=jax.ShapeDtypeStruct((n, HID), jnp.float32),
    )(x, w, b.reshape(1, HID), a.reshape(1))


def _ep_body(a_ref, b_ref, h_ref, d_ref, o_ref, *, relu):
    y = (a_ref[0] + b_ref[0] + h_ref[...]) * d_ref[...]
    if relu:
        y = jnp.maximum(y, 0.0)
    o_ref[...] = y


def _epilogue(acc, h, dinv, relu=False):
    n = h.shape[0]
    return pl.pallas_call(
        functools.partial(_ep_body, relu=relu),
        grid=(n // _BR,),
        in_specs=[
            pl.BlockSpec((1, _BR, HID), lambda i: (0, i, 0)),
            pl.BlockSpec((1, _BR, HID), lambda i: (1, i, 0)),
            pl.BlockSpec((_BR, HID), lambda i: (i, 0)),
            pl.BlockSpec((_BR, 1), lambda i: (i, 0)),
        ],
        out_specs=pl.BlockSpec((_BR, HID), lambda i: (i, 0)),
        out_shape=jax.ShapeDtypeStruct((n, HID), jnp.float32),
    )(acc, acc, h, dinv.reshape(n, 1))


_AGG = {NP1: _make_agg(NP1), NP2: _make_agg(NP2), NP3: _make_agg(NP3)}
_DEG = {NP1: _make_deg(NP1), NP2: _make_deg(NP2), NP3: _make_deg(NP3)}

_Z128 = None


def _pack_edges(src, dst, trash):
    pad = EPAD - E
    srcp = jnp.concatenate([src, jnp.zeros((pad,), jnp.int32)])
    dstp = jnp.concatenate([dst, jnp.full((pad,), trash, jnp.int32)])
    return srcp.reshape(NW, NCH, CH), dstp.reshape(NW, NCH, CH)


def _padr(x, n):
    return jnp.pad(x, ((0, n - x.shape[0]), (0, 0)))


def kernel(feat, edge_index, params):
    p = params
    z128 = jnp.zeros((16, HID), jnp.float32)
    ones_t = jnp.ones((CH, HID), jnp.float32)

    src1 = edge_index[0]
    dst1 = edge_index[1]
    srcp1, dstp1 = _pack_edges(src1, dst1, N)

    def degree(dstp, n_pad):
        d2 = _DEG[n_pad](ones_t, z128, dstp)
        deg = d2[0, :, 0] + d2[1, :, 0]
        return 1.0 / (deg + 1.0)

    def gcn(x, pp, srcp, dstp, dinv, n_pad, relu=False):
        h = _mm(x, pp["W"], pp["b"])
        acc = _AGG[n_pad](h, srcp, dstp, z128)
        return _epilogue(acc, h, dinv, relu=relu)

    def agg_only(h, srcp, dstp, dinv, n_pad):
        acc = _AGG[n_pad](h, srcp, dstp, z128)
        return _epilogue(acc, h, dinv)

    def index_sel(x, x_neg, pp, srcp, dstp, dinv, n_pad, n, k):
        nbp = agg_only(x, srcp, dstp, dinv, n_pad)
        nb = _mm_prelu(nbp, pp["W"], pp["b"], pp["a"])
        pos = jnp.sum(x[:n] * nb[:n], axis=-1)
        neg = jnp.sum(x_neg * nb[:n], axis=-1)
        logit = jnp.concatenate([pos, neg])
        scores = jax.nn.sigmoid(pos)
        _, sel = jax.lax.top_k(scores, k)
        feat_down = x[:n] * scores[:, None]
        return logit, scores, sel, feat_down

    def unpool(xc, sel, pp, srcp, dstp, dinv, n_pad, k):
        hc = _mm_nob(xc, pp["W"])
        hup = (jnp.zeros((n_pad, HID), jnp.float32).at[sel].set(hc[:k])
               + pp["b"][None, :])
        acc = _AGG[n_pad](hup, srcp, dstp, z128)
        return _epilogue(acc, hup, dinv)

    dinv1 = degree(dstp1, NP1)

    w1 = jnp.ones((E,), jnp.float32)

    def _gcn_ref(x, pp, s_, d_, w_, n):
        h = x @ pp["W"] + pp["b"]
        agg = jax.ops.segment_sum(h[s_] * w_[:, None], d_, num_segments=n)
        dg = jax.ops.segment_sum(w_, d_, num_segments=n)
        return (agg + h) / (dg[:, None] + 1.0)

    feat_s1_x = _gcn_ref(feat, p["start_gcn_s1"], src1, dst1, w1, N)
    perm1 = jax.random.permutation(jax.random.key(11), N)
    agg_x = jax.ops.segment_sum(feat_s1_x[src1] * w1[:, None], dst1,
                                num_segments=N)
    deg_x = jax.ops.segment_sum(w1, dst1, num_segments=N)
    nb_x = (agg_x + feat_s1_x) / (deg_x[:, None] + 1.0)
    nb_x = nb_x @ p["is1"]["W"] + p["is1"]["b"]
    nb_x = jnp.where(nb_x > 0, nb_x, p["is1"]["a"] * nb_x)
    pos_x = jnp.sum(feat_s1_x * nb_x, axis=-1)
    neg_x = jnp.sum(feat_s1_x[perm1] * nb_x, axis=-1)
    logit_s1 = jnp.concatenate([pos_x, neg_x])
    scores1 = jax.nn.sigmoid(pos_x)
    _, sel1 = jax.lax.top_k(scores1, K1)
    fd1 = feat_s1_x * scores1[:, None]

    feat_s1 = _padr(feat_s1_x, NP1)
    feat_origin = feat_s1

    new_id1 = jnp.zeros((N,), jnp.int32).at[sel1].set(
        jnp.arange(K1, dtype=jnp.int32))
    valid1 = jnp.zeros((N,), jnp.float32).at[sel1].set(1.0)
    src2 = new_id1[src1]
    dst2r = new_id1[dst1]
    w2 = valid1[src1] * valid1[dst1]
    dst2 = jnp.where(w2 > 0, dst2r, K1)
    srcp2, dstp2 = _pack_edges(src2, dst2, K1)
    dinv2 = degree(dstp2, NP2)

    feat_s2_0 = _padr(feat_s1[sel1] * scores1[sel1][:, None], NP2)
    feat_s2 = gcn(feat_s2_0, p["start_gcn_s2"], srcp2, dstp2, dinv2, NP2)
    perm2 = jax.random.permutation(jax.random.key(12), K1)
    logit_s2, scores2, sel2, fd2 = index_sel(
        feat_s2, feat_s2[:K1][perm2], p["is2"], srcp2, dstp2, dinv2, NP2, K1, K2)

    new_id2 = jnp.zeros((K1,), jnp.int32).at[sel2].set(
        jnp.arange(K2, dtype=jnp.int32))
    valid2 = jnp.zeros((K1,), jnp.float32).at[sel2].set(1.0)
    src3 = new_id2[src2]
    dst3r = new_id2[dst2r]
    w3 = w2 * valid2[src2] * valid2[dst2r]
    dst3 = jnp.where(w3 > 0, dst3r, K2)
    srcp3, dstp3 = _pack_edges(src3, dst3, K2)
    dinv3 = degree(dstp3, NP3)

    feat_s3 = _padr(feat_s2[sel2] * scores2[sel2][:, None], NP3)
    res1, res2, res3 = feat_s1, feat_s2, feat_s3

    f1 = gcn(feat_s1, p["s1_l1"], srcp1, dstp1, dinv1, NP1, relu=True)
    f2 = gcn(feat_s2, p["s2_l1"], srcp2, dstp2, dinv2, NP2, relu=True)
    f3 = gcn(feat_s3, p["s3_l1"], srcp3, dstp3, dinv3, NP3, relu=True)

    def pool(x, pp, srcp, dstp, dinv, n_pad, sel, scores, np_out):
        g = gcn(x, pp, srcp, dstp, dinv, n_pad)
        return _padr(g[sel] * scores[sel][:, None], np_out)

    f12 = pool(f1, p["pool_s12_1"], srcp1, dstp1, dinv1, NP1, sel1, scores1, NP2)
    f21 = unpool(f2, sel1, p["unpool_s21_1"], srcp1, dstp1, dinv1, NP1, K1)
    f23 = pool(f2, p["pool_s23_1"], srcp2, dstp2, dinv2, NP2, sel2, scores2, NP3)
    f32 = unpool(f3, sel2, p["unpool_s32_1"], srcp2, dstp2, dinv2, NP2, K2)
    f1 = f1 + f21 + res1
    f2 = f2 + (f12 + f32) * 0.5 + res2
    f3 = f3 + f23 + res3

    f1 = gcn(f1, p["s1_l2"], srcp1, dstp1, dinv1, NP1, relu=True)
    f2 = gcn(f2, p["s2_l2"], srcp2, dstp2, dinv2, NP2, relu=True)
    f3 = gcn(f3, p["s3_l2"], srcp3, dstp3, dinv3, NP3, relu=True)

    f12 = pool(f1, p["pool_s12_2"], srcp1, dstp1, dinv1, NP1, sel1, scores1, NP2)
    f21 = unpool(f2, sel1, p["unpool_s21_2"], srcp1, dstp1, dinv1, NP1, K1)
    f23 = pool(f2, p["pool_s23_2"], srcp2, dstp2, dinv2, NP2, sel2, scores2, NP3)
    f32 = unpool(f3, sel2, p["unpool_s32_2"], srcp2, dstp2, dinv2, NP2, K2)
    cw = 0.05
    f1 = f1 + cw * f21
    f2 = f2 + cw * (f12 + f32) * 0.5
    f3 = f3 + cw * f23

    f1 = gcn(f1, p["s1_l3"], srcp1, dstp1, dinv1, NP1, relu=True)
    f2 = gcn(f2, p["s2_l3"], srcp2, dstp2, dinv2, NP2, relu=True)
    f3 = gcn(f3, p["s3_l3"], srcp3, dstp3, dinv3, NP3, relu=True)

    feat_s3_out = (unpool(f3, sel2, p["unpool_end_s32"], srcp2, dstp2, dinv2,
                          NP2, K2)
                   + _padr(fd2, NP2))
    feat_s2_out = unpool(f2 + feat_s3_out, sel1, p["unpool_end_s21"],
                         srcp1, dstp1, dinv1, NP1, K1)
    feat_agg = f1 + feat_s2_out + _padr(fd1, NP1)
    feat_cat = jnp.concatenate([feat_agg, feat_origin], axis=1)
    feat_agg = gcn(feat_cat, p["end_gcn"], srcp1, dstp1, dinv1, NP1)
    return feat_agg[:N], logit_s1, logit_s2
